# initial kernel scaffold (unmeasured)
import jax
import jax.numpy as jnp
from jax import lax
from jax.experimental import pallas as pl
from jax.experimental.pallas import tpu as pltpu

N_DEV = 4
BQ = 2
HG = 4
SQ = 256
DH = 64
DM = 512
DQ = 256


def kernel(x, Wq, K_ext, V_ext, Wo):
    def body(x_ref, wq_ref, k_ref, v_ref, wo_ref, out_ref,
             wq_comm, wo_comm, kscr, vscr,
             wq_ssem, wq_rsem, wo_ssem, wo_rsem, ksem, vsem):
        my = lax.axis_index("i")
        left = lax.rem(my + N_DEV - 1, N_DEV)
        right = lax.rem(my + 1, N_DEV)
        b0 = my * BQ

        bar = pltpu.get_barrier_semaphore()
        for nbr in (left, right):
            pl.semaphore_signal(bar, inc=1, device_id=(nbr,),
                                device_id_type=pl.DeviceIdType.MESH)
        pl.semaphore_wait(bar, 2)

        wq_comm[0] = wq_ref[...].astype(jnp.bfloat16)
        wo_comm[0] = wo_ref[...].astype(jnp.bfloat16)

        kv_waits = []
        for h in range(N_DEV):
            gi = lax.rem(my + N_DEV - h, N_DEV)
            group = []
            for b in range(BQ):
                for hh in range(HG):
                    ck = pltpu.make_async_copy(
                        k_ref.at[b0 + b, :, gi * HG + hh, :],
                        kscr.at[h, b, hh], ksem.at[h, b, hh])
                    cv = pltpu.make_async_copy(
                        v_ref.at[b0 + b, :, gi * HG + hh, :],
                        vscr.at[h, b, hh], vsem.at[h, b, hh])
                    ck.start()
                    cv.start()
                    group.append((ck, cv))
            kv_waits.append(group)

        def hop(h):
            mk = lambda src, ssem, rsem: pltpu.make_async_remote_copy(
                src_ref=src.at[h], dst_ref=src.at[h + 1],
                send_sem=ssem.at[h], recv_sem=rsem.at[h],
                device_id=(right,), device_id_type=pl.DeviceIdType.MESH)
            return (mk(wq_comm, wq_ssem, wq_rsem),
                    mk(wo_comm, wo_ssem, wo_rsem))

        rdmas = [hop(h) for h in range(N_DEV - 1)]

        qb = lax.broadcasted_iota(jnp.int32, (SQ, SQ), 0) // 64
        kb = lax.broadcasted_iota(jnp.int32, (SQ, SQ), 1) // 64
        mask = (qb == kb) | ((kb % 4) == (qb % 4))

        xb16 = [x_ref[b].astype(jnp.bfloat16) for b in range(BQ)]

        def compute_group(h):
            for ck, cv in kv_waits[h]:
                ck.wait()
                cv.wait()
            wqh = wq_comm[h]
            woh = wo_comm[h]
            for b in range(BQ):
                q = jnp.dot(xb16[b], wqh,
                            preferred_element_type=jnp.float32)
                ctx = []
                for hh in range(HG):
                    qh = q[:, hh * DH:(hh + 1) * DH].astype(jnp.bfloat16)
                    kh = kscr[h, b, hh].astype(jnp.bfloat16)
                    s = lax.dot_general(
                        qh, kh, (((1,), (1,)), ((), ())),
                        preferred_element_type=jnp.float32) * 0.125
                    s = jnp.where(mask, s, -1e9)
                    e = jnp.exp(s - jnp.max(s, axis=1, keepdims=True))
                    w = (e / jnp.sum(e, axis=1, keepdims=True)
                         ).astype(jnp.bfloat16)
                    vh = vscr[h, b, hh].astype(jnp.bfloat16)
                    ctx.append(jnp.dot(w, vh,
                                       preferred_element_type=jnp.float32))
                ctxc = jnp.concatenate(ctx, axis=1).astype(jnp.bfloat16)
                contrib = jnp.dot(ctxc, woh,
                                  preferred_element_type=jnp.float32)
                if h == 0:
                    out_ref[b] = contrib
                else:
                    out_ref[b] = out_ref[b] + contrib

        rdmas[0][0].start()
        rdmas[0][1].start()
        compute_group(0)
        for h in range(1, N_DEV):
            rdmas[h - 1][0].wait()
            rdmas[h - 1][1].wait()
            if h < N_DEV - 1:
                rdmas[h][0].start()
                rdmas[h][1].start()
            compute_group(h)

    out_shape = jax.ShapeDtypeStruct((BQ, SQ, DM), jnp.float32)
    return pl.pallas_call(
        body,
        out_shape=out_shape,
        in_specs=[
            pl.BlockSpec(memory_space=pltpu.VMEM),
            pl.BlockSpec(memory_space=pltpu.VMEM),
            pl.BlockSpec(memory_space=pltpu.ANY),
            pl.BlockSpec(memory_space=pltpu.ANY),
            pl.BlockSpec(memory_space=pltpu.VMEM),
        ],
        out_specs=pl.BlockSpec(memory_space=pltpu.VMEM),
        scratch_shapes=[
            pltpu.VMEM((N_DEV, DM, DQ), jnp.bfloat16),
            pltpu.VMEM((N_DEV, DQ, DM), jnp.bfloat16),
            pltpu.VMEM((N_DEV, BQ, HG, SQ, DH), jnp.float32),
            pltpu.VMEM((N_DEV, BQ, HG, SQ, DH), jnp.float32),
            pltpu.SemaphoreType.DMA((N_DEV - 1,)),
            pltpu.SemaphoreType.DMA((N_DEV - 1,)),
            pltpu.SemaphoreType.DMA((N_DEV - 1,)),
            pltpu.SemaphoreType.DMA((N_DEV - 1,)),
            pltpu.SemaphoreType.DMA((N_DEV, BQ, HG)),
            pltpu.SemaphoreType.DMA((N_DEV, BQ, HG)),
        ],
        compiler_params=pltpu.CompilerParams(collective_id=0),
    )(x, Wq, K_ext, V_ext, Wo)


# baseline (device time: 56445 ns/iter reference)
import jax
import jax.numpy as jnp
from jax import lax
from jax.experimental import pallas as pl
from jax.experimental.pallas import tpu as pltpu

N_DEV = 4
BQ = 2
HG = 4
SQ = 256
DH = 64
DM = 512
DQ = 256


def kernel(x, Wq, K_ext, V_ext, Wo):
    def body(x_ref, wq_ref, k_ref, v_ref, wo_ref, out_ref,
             wq_comm, wo_comm, kscr, vscr,
             wq_ssem, wq_rsem, wo_ssem, wo_rsem, ksem, vsem):
        my = lax.axis_index("i")
        left = lax.rem(my + N_DEV - 1, N_DEV)
        right = lax.rem(my + 1, N_DEV)
        b0 = my * BQ

        bar = pltpu.get_barrier_semaphore()
        for nbr in (left, right):
            pl.semaphore_signal(bar, inc=1, device_id=(nbr,),
                                device_id_type=pl.DeviceIdType.MESH)
        pl.semaphore_wait(bar, 2)

        wq_comm[0] = wq_ref[...].astype(jnp.bfloat16)
        wo_comm[0] = wo_ref[...].astype(jnp.bfloat16)

        kv_waits = []
        for h in range(N_DEV):
            gi = lax.rem(my + N_DEV - h, N_DEV)
            group = []
            for b in range(BQ):
                for hh in range(HG):
                    ck = pltpu.make_async_copy(
                        k_ref.at[b0 + b, :, gi * HG + hh, :],
                        kscr.at[h, b, hh], ksem.at[h, b, hh])
                    cv = pltpu.make_async_copy(
                        v_ref.at[b0 + b, :, gi * HG + hh, :],
                        vscr.at[h, b, hh], vsem.at[h, b, hh])
                    ck.start()
                    cv.start()
                    group.append((ck, cv))
            kv_waits.append(group)

        def hop(h):
            mk = lambda src, ssem, rsem: pltpu.make_async_remote_copy(
                src_ref=src.at[h], dst_ref=src.at[h + 1],
                send_sem=ssem.at[h], recv_sem=rsem.at[h],
                device_id=(right,), device_id_type=pl.DeviceIdType.MESH)
            return (mk(wq_comm, wq_ssem, wq_rsem),
                    mk(wo_comm, wo_ssem, wo_rsem))

        rdmas = [hop(h) for h in range(N_DEV - 1)]

        qb = lax.broadcasted_iota(jnp.int32, (SQ, SQ), 0) // 64
        kb = lax.broadcasted_iota(jnp.int32, (SQ, SQ), 1) // 64
        mask = (qb == kb) | ((kb % 4) == (qb % 4))

        xb16 = [x_ref[b].astype(jnp.bfloat16) for b in range(BQ)]

        def compute_group(h):
            for ck, cv in kv_waits[h]:
                ck.wait()
                cv.wait()
            wqh = wq_comm[h]
            woh = wo_comm[h]
            for b in range(BQ):
                q = jnp.dot(xb16[b], wqh,
                            preferred_element_type=jnp.float32)
                ctx = []
                for hh in range(HG):
                    qh = q[:, hh * DH:(hh + 1) * DH].astype(jnp.bfloat16)
                    kh = kscr[h, b, hh].astype(jnp.bfloat16)
                    s = lax.dot_general(
                        qh, kh, (((1,), (1,)), ((), ())),
                        preferred_element_type=jnp.float32) * 0.125
                    s = jnp.where(mask, s, -1e9)
                    e = jnp.exp(s - jnp.max(s, axis=1, keepdims=True))
                    w = (e / jnp.sum(e, axis=1, keepdims=True)
                         ).astype(jnp.bfloat16)
                    vh = vscr[h, b, hh].astype(jnp.bfloat16)
                    ctx.append(jnp.dot(w, vh,
                                       preferred_element_type=jnp.float32))
                ctxc = jnp.concatenate(ctx, axis=1).astype(jnp.bfloat16)
                contrib = jnp.dot(ctxc, woh,
                                  preferred_element_type=jnp.float32)
                if h == 0:
                    out_ref[b] = contrib
                else:
                    out_ref[b] = out_ref[b] + contrib

        rdmas[0][0].start()
        rdmas[0][1].start()
        compute_group(0)
        for h in range(1, N_DEV):
            rdmas[h - 1][0].wait()
            rdmas[h - 1][1].wait()
            if h < N_DEV - 1:
                rdmas[h][0].start()
                rdmas[h][1].start()
            compute_group(h)

    out_shape = jax.ShapeDtypeStruct((BQ, SQ, DM), jnp.float32)
    return pl.pallas_call(
        body,
        out_shape=out_shape,
        in_specs=[
            pl.BlockSpec(memory_space=pltpu.VMEM),
            pl.BlockSpec(memory_space=pltpu.VMEM),
            pl.BlockSpec(memory_space=pl.ANY),
            pl.BlockSpec(memory_space=pl.ANY),
            pl.BlockSpec(memory_space=pltpu.VMEM),
        ],
        out_specs=pl.BlockSpec(memory_space=pltpu.VMEM),
        scratch_shapes=[
            pltpu.VMEM((N_DEV, DM, DQ), jnp.bfloat16),
            pltpu.VMEM((N_DEV, DQ, DM), jnp.bfloat16),
            pltpu.VMEM((N_DEV, BQ, HG, SQ, DH), jnp.float32),
            pltpu.VMEM((N_DEV, BQ, HG, SQ, DH), jnp.float32),
            pltpu.SemaphoreType.DMA((N_DEV - 1,)),
            pltpu.SemaphoreType.DMA((N_DEV - 1,)),
            pltpu.SemaphoreType.DMA((N_DEV - 1,)),
            pltpu.SemaphoreType.DMA((N_DEV - 1,)),
            pltpu.SemaphoreType.DMA((N_DEV, BQ, HG)),
            pltpu.SemaphoreType.DMA((N_DEV, BQ, HG)),
        ],
        compiler_params=pltpu.CompilerParams(collective_id=0),
    )(x, Wq, K_ext, V_ext, Wo)


# device time: 51513 ns/iter; 1.0957x vs baseline; 1.0957x over previous
import jax
import jax.numpy as jnp
from jax import lax
from jax.experimental import pallas as pl
from jax.experimental.pallas import tpu as pltpu

N_DEV = 4
BQ = 2
HG = 4
SQ = 256
DH = 64
DM = 512
DQ = 256


def kernel(x, Wq, K_ext, V_ext, Wo):
    def body(x_ref, wq_ref, k_ref, v_ref, wo_ref, out_ref,
             wq_comm, wo_comm, kscr, vscr,
             wq_ssem, wq_rsem, wo_ssem, wo_rsem, ksem, vsem):
        my = lax.axis_index("i")
        left = lax.rem(my + N_DEV - 1, N_DEV)
        right = lax.rem(my + 1, N_DEV)
        b0 = my * BQ

        bar = pltpu.get_barrier_semaphore()
        for nbr in (left, right):
            pl.semaphore_signal(bar, inc=1, device_id=(nbr,),
                                device_id_type=pl.DeviceIdType.MESH)
        pl.semaphore_wait(bar, 2)

        wq_comm[0] = wq_ref[...].astype(jnp.bfloat16)
        wo_comm[0] = wo_ref[...].astype(jnp.bfloat16)

        BISECT = 2

        kv_waits = []
        for h in range(N_DEV if BISECT < 2 else 0):
            gi = lax.rem(my + N_DEV - h, N_DEV)
            group = []
            for b in range(BQ):
                for hh in range(HG):
                    ck = pltpu.make_async_copy(
                        k_ref.at[b0 + b, :, gi * HG + hh, :],
                        kscr.at[h, b, hh], ksem.at[h, b, hh])
                    cv = pltpu.make_async_copy(
                        v_ref.at[b0 + b, :, gi * HG + hh, :],
                        vscr.at[h, b, hh], vsem.at[h, b, hh])
                    ck.start()
                    cv.start()
                    group.append((ck, cv))
            kv_waits.append(group)

        def hop(h):
            mk = lambda src, ssem, rsem: pltpu.make_async_remote_copy(
                src_ref=src.at[h], dst_ref=src.at[h + 1],
                send_sem=ssem.at[h], recv_sem=rsem.at[h],
                device_id=(right,), device_id_type=pl.DeviceIdType.MESH)
            return (mk(wq_comm, wq_ssem, wq_rsem),
                    mk(wo_comm, wo_ssem, wo_rsem))

        rdmas = [hop(h) for h in range(N_DEV - 1)]

        qb = lax.broadcasted_iota(jnp.int32, (SQ, SQ), 0) // 64
        kb = lax.broadcasted_iota(jnp.int32, (SQ, SQ), 1) // 64
        mask = (qb == kb) | ((kb % 4) == (qb % 4))

        xb16 = [x_ref[b].astype(jnp.bfloat16) for b in range(BQ)]

        def compute_group(h):
            if BISECT < 2:
                for ck, cv in kv_waits[h]:
                    ck.wait()
                    cv.wait()
            wqh = wq_comm[h]
            woh = wo_comm[h]
            for b in range(BQ):
                q = jnp.dot(xb16[b], wqh,
                            preferred_element_type=jnp.float32)
                if BISECT == 2:
                    ctxc = q.astype(jnp.bfloat16)
                else:
                    ctx = []
                    for hh in range(HG):
                        qh = q[:, hh * DH:(hh + 1) * DH].astype(jnp.bfloat16)
                        kh = kscr[h, b, hh].astype(jnp.bfloat16)
                        s = lax.dot_general(
                            qh, kh, (((1,), (1,)), ((), ())),
                            preferred_element_type=jnp.float32) * 0.125
                        if BISECT == 1:
                            w = (s * 0.001).astype(jnp.bfloat16)
                        else:
                            s = jnp.where(mask, s, -1e9)
                            e = jnp.exp(s - jnp.max(s, axis=1, keepdims=True))
                            w = (e / jnp.sum(e, axis=1, keepdims=True)
                                 ).astype(jnp.bfloat16)
                        vh = vscr[h, b, hh].astype(jnp.bfloat16)
                        ctx.append(jnp.dot(w, vh,
                                           preferred_element_type=jnp.float32))
                    ctxc = jnp.concatenate(ctx, axis=1).astype(jnp.bfloat16)
                contrib = jnp.dot(ctxc, woh,
                                  preferred_element_type=jnp.float32)
                if h == 0:
                    out_ref[b] = contrib
                else:
                    out_ref[b] = out_ref[b] + contrib

        rdmas[0][0].start()
        rdmas[0][1].start()
        compute_group(0)
        for h in range(1, N_DEV):
            rdmas[h - 1][0].wait()
            rdmas[h - 1][1].wait()
            if h < N_DEV - 1:
                rdmas[h][0].start()
                rdmas[h][1].start()
            compute_group(h)

    out_shape = jax.ShapeDtypeStruct((BQ, SQ, DM), jnp.float32)
    return pl.pallas_call(
        body,
        out_shape=out_shape,
        in_specs=[
            pl.BlockSpec(memory_space=pltpu.VMEM),
            pl.BlockSpec(memory_space=pltpu.VMEM),
            pl.BlockSpec(memory_space=pl.ANY),
            pl.BlockSpec(memory_space=pl.ANY),
            pl.BlockSpec(memory_space=pltpu.VMEM),
        ],
        out_specs=pl.BlockSpec(memory_space=pltpu.VMEM),
        scratch_shapes=[
            pltpu.VMEM((N_DEV, DM, DQ), jnp.bfloat16),
            pltpu.VMEM((N_DEV, DQ, DM), jnp.bfloat16),
            pltpu.VMEM((N_DEV, BQ, HG, SQ, DH), jnp.float32),
            pltpu.VMEM((N_DEV, BQ, HG, SQ, DH), jnp.float32),
            pltpu.SemaphoreType.DMA((N_DEV - 1,)),
            pltpu.SemaphoreType.DMA((N_DEV - 1,)),
            pltpu.SemaphoreType.DMA((N_DEV - 1,)),
            pltpu.SemaphoreType.DMA((N_DEV - 1,)),
            pltpu.SemaphoreType.DMA((N_DEV, BQ, HG)),
            pltpu.SemaphoreType.DMA((N_DEV, BQ, HG)),
        ],
        compiler_params=pltpu.CompilerParams(collective_id=0),
    )(x, Wq, K_ext, V_ext, Wo)


# device time: 37791 ns/iter; 1.4936x vs baseline; 1.3631x over previous
import jax
import jax.numpy as jnp
from jax import lax
from jax.experimental import pallas as pl
from jax.experimental.pallas import tpu as pltpu

N_DEV = 4
BQ = 2
HG = 4
SQ = 256
DH = 64
DM = 512
DQ = 256


def kernel(x, Wq, K_ext, V_ext, Wo):
    def body(x_ref, wq_ref, k_ref, v_ref, wo_ref, out_ref,
             wq_comm, wo_comm, kscr, vscr,
             wq_ssem, wq_rsem, wo_ssem, wo_rsem, ksem, vsem):
        my = lax.axis_index("i")
        left = lax.rem(my + N_DEV - 1, N_DEV)
        right = lax.rem(my + 1, N_DEV)
        b0 = my * BQ

        bar = pltpu.get_barrier_semaphore()
        for nbr in (left, right):
            pl.semaphore_signal(bar, inc=1, device_id=(nbr,),
                                device_id_type=pl.DeviceIdType.MESH)
        pl.semaphore_wait(bar, 2)

        wq_comm[0] = wq_ref[...].astype(jnp.bfloat16)
        wo_comm[0] = wo_ref[...].astype(jnp.bfloat16)

        BISECT = 3

        kv_waits = []
        for h in range(N_DEV if BISECT < 2 else 0):
            gi = lax.rem(my + N_DEV - h, N_DEV)
            group = []
            for b in range(BQ):
                for hh in range(HG):
                    ck = pltpu.make_async_copy(
                        k_ref.at[b0 + b, :, gi * HG + hh, :],
                        kscr.at[h, b, hh], ksem.at[h, b, hh])
                    cv = pltpu.make_async_copy(
                        v_ref.at[b0 + b, :, gi * HG + hh, :],
                        vscr.at[h, b, hh], vsem.at[h, b, hh])
                    ck.start()
                    cv.start()
                    group.append((ck, cv))
            kv_waits.append(group)

        def hop(h):
            mk = lambda src, ssem, rsem: pltpu.make_async_remote_copy(
                src_ref=src.at[h], dst_ref=src.at[h + 1],
                send_sem=ssem.at[h], recv_sem=rsem.at[h],
                device_id=(right,), device_id_type=pl.DeviceIdType.MESH)
            return (mk(wq_comm, wq_ssem, wq_rsem),
                    mk(wo_comm, wo_ssem, wo_rsem))

        rdmas = [hop(h) for h in range(N_DEV - 1)]

        qb = lax.broadcasted_iota(jnp.int32, (SQ, SQ), 0) // 64
        kb = lax.broadcasted_iota(jnp.int32, (SQ, SQ), 1) // 64
        mask = (qb == kb) | ((kb % 4) == (qb % 4))

        xb16 = [x_ref[b].astype(jnp.bfloat16) for b in range(BQ)]

        def compute_group(h):
            if BISECT < 2:
                for ck, cv in kv_waits[h]:
                    ck.wait()
                    cv.wait()
            wqh = wq_comm[h]
            woh = wo_comm[h]
            for b in range(BQ):
                q = jnp.dot(xb16[b], wqh,
                            preferred_element_type=jnp.float32)
                if BISECT == 2:
                    ctxc = q.astype(jnp.bfloat16)
                else:
                    ctx = []
                    for hh in range(HG):
                        qh = q[:, hh * DH:(hh + 1) * DH].astype(jnp.bfloat16)
                        kh = kscr[h, b, hh].astype(jnp.bfloat16)
                        s = lax.dot_general(
                            qh, kh, (((1,), (1,)), ((), ())),
                            preferred_element_type=jnp.float32) * 0.125
                        if BISECT == 1:
                            w = (s * 0.001).astype(jnp.bfloat16)
                        else:
                            s = jnp.where(mask, s, -1e9)
                            e = jnp.exp(s - jnp.max(s, axis=1, keepdims=True))
                            w = (e / jnp.sum(e, axis=1, keepdims=True)
                                 ).astype(jnp.bfloat16)
                        vh = vscr[h, b, hh].astype(jnp.bfloat16)
                        ctx.append(jnp.dot(w, vh,
                                           preferred_element_type=jnp.float32))
                    ctxc = jnp.concatenate(ctx, axis=1).astype(jnp.bfloat16)
                contrib = jnp.dot(ctxc, woh,
                                  preferred_element_type=jnp.float32)
                if h == 0:
                    out_ref[b] = contrib
                else:
                    out_ref[b] = out_ref[b] + contrib

        if BISECT >= 3:
            for h in range(N_DEV):
                compute_group(0)
        else:
            rdmas[0][0].start()
            rdmas[0][1].start()
            compute_group(0)
            for h in range(1, N_DEV):
                rdmas[h - 1][0].wait()
                rdmas[h - 1][1].wait()
                if h < N_DEV - 1:
                    rdmas[h][0].start()
                    rdmas[h][1].start()
                compute_group(h)

    out_shape = jax.ShapeDtypeStruct((BQ, SQ, DM), jnp.float32)
    return pl.pallas_call(
        body,
        out_shape=out_shape,
        in_specs=[
            pl.BlockSpec(memory_space=pltpu.VMEM),
            pl.BlockSpec(memory_space=pltpu.VMEM),
            pl.BlockSpec(memory_space=pl.ANY),
            pl.BlockSpec(memory_space=pl.ANY),
            pl.BlockSpec(memory_space=pltpu.VMEM),
        ],
        out_specs=pl.BlockSpec(memory_space=pltpu.VMEM),
        scratch_shapes=[
            pltpu.VMEM((N_DEV, DM, DQ), jnp.bfloat16),
            pltpu.VMEM((N_DEV, DQ, DM), jnp.bfloat16),
            pltpu.VMEM((N_DEV, BQ, HG, SQ, DH), jnp.float32),
            pltpu.VMEM((N_DEV, BQ, HG, SQ, DH), jnp.float32),
            pltpu.SemaphoreType.DMA((N_DEV - 1,)),
            pltpu.SemaphoreType.DMA((N_DEV - 1,)),
            pltpu.SemaphoreType.DMA((N_DEV - 1,)),
            pltpu.SemaphoreType.DMA((N_DEV - 1,)),
            pltpu.SemaphoreType.DMA((N_DEV, BQ, HG)),
            pltpu.SemaphoreType.DMA((N_DEV, BQ, HG)),
        ],
        compiler_params=pltpu.CompilerParams(collective_id=0),
    )(x, Wq, K_ext, V_ext, Wo)


# device time: 29912 ns/iter; 1.8870x vs baseline; 1.2634x over previous
import jax
import jax.numpy as jnp
from jax import lax
from jax.experimental import pallas as pl
from jax.experimental.pallas import tpu as pltpu

N_DEV = 4
BQ = 2
HG = 4
SQ = 256
DH = 64
DM = 512
DQ = 256

_MESH = pl.DeviceIdType.MESH


def kernel(x, Wq, K_ext, V_ext, Wo):
    kT = jnp.transpose(K_ext, (0, 2, 3, 1))
    vT = jnp.transpose(V_ext, (0, 2, 3, 1))

    def body(x_ref, wq_ref, kt_ref, vt_ref, wo_ref, out_ref,
             wq_all, wo_all, kts, vts,
             ssem, rsem, ksem, vsem):
        my = lax.axis_index("i")
        left = lax.rem(my + N_DEV - 1, N_DEV)
        right = lax.rem(my + 1, N_DEV)
        b0 = my * BQ

        bar = pltpu.get_barrier_semaphore()
        for nbr in (left, right):
            pl.semaphore_signal(bar, inc=1, device_id=(nbr,),
                                device_id_type=_MESH)
        pl.semaphore_wait(bar, 2)

        slot_g = [my, left, right, lax.rem(my + 2, N_DEV)]

        kv_waits = []
        for s in range(N_DEV):
            g4 = slot_g[s] * HG
            group = []
            for b in range(BQ):
                ck = pltpu.make_async_copy(
                    kt_ref.at[b0 + b, pl.ds(g4, HG)],
                    kts.at[s, b], ksem.at[s, b])
                cv = pltpu.make_async_copy(
                    vt_ref.at[b0 + b, pl.ds(g4, HG)],
                    vts.at[s, b], vsem.at[s, b])
                ck.start()
                cv.start()
                group.append((ck, cv))
            kv_waits.append(group)

        wq_all[0] = wq_ref[...].astype(jnp.bfloat16)
        wo_all[0] = wo_ref[...].astype(jnp.bfloat16)

        def rcopy(i, src, dst, dev):
            return pltpu.make_async_remote_copy(
                src_ref=src, dst_ref=dst, send_sem=ssem.at[i],
                recv_sem=rsem.at[i], device_id=(dev,), device_id_type=_MESH)

        a_wq = rcopy(0, wq_all.at[0], wq_all.at[2], left)
        a_wo = rcopy(1, wo_all.at[0], wo_all.at[2], left)
        b_wq = rcopy(2, wq_all.at[0], wq_all.at[1], right)
        b_wo = rcopy(3, wo_all.at[0], wo_all.at[1], right)
        c_wq = rcopy(4, wq_all.at[2, pl.ds(0, DM // 2)],
                     wq_all.at[3, pl.ds(0, DM // 2)], left)
        c_wo = rcopy(5, wo_all.at[2, pl.ds(0, DQ // 2)],
                     wo_all.at[3, pl.ds(0, DQ // 2)], left)
        d_wq = rcopy(6, wq_all.at[1, pl.ds(DM // 2, DM // 2)],
                     wq_all.at[3, pl.ds(DM // 2, DM // 2)], right)
        d_wo = rcopy(7, wo_all.at[1, pl.ds(DQ // 2, DQ // 2)],
                     wo_all.at[3, pl.ds(DQ // 2, DQ // 2)], right)

        qb = lax.broadcasted_iota(jnp.int32, (SQ, SQ), 0) // 64
        kb = lax.broadcasted_iota(jnp.int32, (SQ, SQ), 1) // 64
        mask = (qb == kb) | ((kb % 4) == (qb % 4))

        xb16 = [x_ref[b].astype(jnp.bfloat16) for b in range(BQ)]

        def compute_slot(s):
            for ck, cv in kv_waits[s]:
                ck.wait()
                cv.wait()
            wqh = wq_all[s]
            woh = wo_all[s]
            for b in range(BQ):
                q = jnp.dot(xb16[b], wqh,
                            preferred_element_type=jnp.float32)
                ctx = []
                for hh in range(HG):
                    qh = q[:, hh * DH:(hh + 1) * DH].astype(jnp.bfloat16)
                    kth = kts[s, b, hh].astype(jnp.bfloat16)
                    sc = jnp.dot(qh, kth,
                                 preferred_element_type=jnp.float32) * 0.125
                    sc = jnp.where(mask, sc, -1e9)
                    e = jnp.exp(sc - jnp.max(sc, axis=1, keepdims=True))
                    w = (e / jnp.sum(e, axis=1, keepdims=True)
                         ).astype(jnp.bfloat16)
                    vth = vts[s, b, hh].astype(jnp.bfloat16)
                    ctx.append(lax.dot_general(
                        w, vth, (((1,), (1,)), ((), ())),
                        preferred_element_type=jnp.float32))
                ctxc = jnp.concatenate(ctx, axis=1).astype(jnp.bfloat16)
                contrib = jnp.dot(ctxc, woh,
                                  preferred_element_type=jnp.float32)
                if s == 0:
                    out_ref[b] = contrib
                else:
                    out_ref[b] = out_ref[b] + contrib

        for r in (a_wq, a_wo, b_wq, b_wo):
            r.start()
        compute_slot(0)
        for r in (a_wq, a_wo, b_wq, b_wo):
            r.wait()
        for r in (c_wq, c_wo, d_wq, d_wo):
            r.start()
        compute_slot(1)
        compute_slot(2)
        for r in (c_wq, c_wo, d_wq, d_wo):
            r.wait()
        compute_slot(3)

    out_shape = jax.ShapeDtypeStruct((BQ, SQ, DM), jnp.float32)
    return pl.pallas_call(
        body,
        out_shape=out_shape,
        in_specs=[
            pl.BlockSpec(memory_space=pltpu.VMEM),
            pl.BlockSpec(memory_space=pltpu.VMEM),
            pl.BlockSpec(memory_space=pl.ANY),
            pl.BlockSpec(memory_space=pl.ANY),
            pl.BlockSpec(memory_space=pltpu.VMEM),
        ],
        out_specs=pl.BlockSpec(memory_space=pltpu.VMEM),
        scratch_shapes=[
            pltpu.VMEM((N_DEV, DM, DQ), jnp.bfloat16),
            pltpu.VMEM((N_DEV, DQ, DM), jnp.bfloat16),
            pltpu.VMEM((N_DEV, BQ, HG, DH, SQ), jnp.float32),
            pltpu.VMEM((N_DEV, BQ, HG, DH, SQ), jnp.float32),
            pltpu.SemaphoreType.DMA((8,)),
            pltpu.SemaphoreType.DMA((8,)),
            pltpu.SemaphoreType.DMA((N_DEV, BQ)),
            pltpu.SemaphoreType.DMA((N_DEV, BQ)),
        ],
        compiler_params=pltpu.CompilerParams(collective_id=0),
    )(x, Wq, kT, vT, Wo)


# device time: 22472 ns/iter; 2.5118x vs baseline; 1.3311x over previous
import jax
import jax.numpy as jnp
from jax import lax
from jax.experimental import pallas as pl
from jax.experimental.pallas import tpu as pltpu

N_DEV = 4
BQ = 2
HG = 4
SQ = 256
DH = 64
DM = 512
DQ = 256

_MESH = pl.DeviceIdType.MESH


def kernel(x, Wq, K_ext, V_ext, Wo):
    b0 = lax.axis_index("i") * BQ
    kT = jnp.transpose(
        lax.dynamic_slice_in_dim(K_ext, b0, BQ, axis=0), (0, 2, 3, 1))
    vT = jnp.transpose(
        lax.dynamic_slice_in_dim(V_ext, b0, BQ, axis=0), (0, 2, 3, 1))

    def body(x_ref, wq_ref, kt_ref, vt_ref, wo_ref, out_ref,
             wq_all, wo_all, kts, vts,
             ssem, rsem, ksem, vsem):
        my = lax.axis_index("i")
        left = lax.rem(my + N_DEV - 1, N_DEV)
        right = lax.rem(my + 1, N_DEV)

        bar = pltpu.get_barrier_semaphore()
        for nbr in (left, right):
            pl.semaphore_signal(bar, inc=1, device_id=(nbr,),
                                device_id_type=_MESH)
        pl.semaphore_wait(bar, 2)

        slot_g = [my, left, right, lax.rem(my + 2, N_DEV)]

        kv_waits = []
        for s in range(N_DEV):
            g4 = slot_g[s] * HG
            group = []
            for b in range(BQ):
                ck = pltpu.make_async_copy(
                    kt_ref.at[b, pl.ds(g4, HG)],
                    kts.at[s, b], ksem.at[s, b])
                cv = pltpu.make_async_copy(
                    vt_ref.at[b, pl.ds(g4, HG)],
                    vts.at[s, b], vsem.at[s, b])
                ck.start()
                cv.start()
                group.append((ck, cv))
            kv_waits.append(group)

        wq_all[0] = wq_ref[...].astype(jnp.bfloat16)
        wo_all[0] = wo_ref[...].astype(jnp.bfloat16)

        def rcopy(i, src, dst, dev):
            return pltpu.make_async_remote_copy(
                src_ref=src, dst_ref=dst, send_sem=ssem.at[i],
                recv_sem=rsem.at[i], device_id=(dev,), device_id_type=_MESH)

        a_wq = rcopy(0, wq_all.at[0], wq_all.at[2], left)
        a_wo = rcopy(1, wo_all.at[0], wo_all.at[2], left)
        b_wq = rcopy(2, wq_all.at[0], wq_all.at[1], right)
        b_wo = rcopy(3, wo_all.at[0], wo_all.at[1], right)
        c_wq = rcopy(4, wq_all.at[2, pl.ds(0, DM // 2)],
                     wq_all.at[3, pl.ds(0, DM // 2)], left)
        c_wo = rcopy(5, wo_all.at[2, pl.ds(0, DQ // 2)],
                     wo_all.at[3, pl.ds(0, DQ // 2)], left)
        d_wq = rcopy(6, wq_all.at[1, pl.ds(DM // 2, DM // 2)],
                     wq_all.at[3, pl.ds(DM // 2, DM // 2)], right)
        d_wo = rcopy(7, wo_all.at[1, pl.ds(DQ // 2, DQ // 2)],
                     wo_all.at[3, pl.ds(DQ // 2, DQ // 2)], right)

        qb = lax.broadcasted_iota(jnp.int32, (SQ, SQ), 0) // 64
        kb = lax.broadcasted_iota(jnp.int32, (SQ, SQ), 1) // 64
        mask = (qb == kb) | ((kb % 4) == (qb % 4))

        xb16 = [x_ref[b].astype(jnp.bfloat16) for b in range(BQ)]

        def compute_slot(s):
            for ck, cv in kv_waits[s]:
                ck.wait()
                cv.wait()
            wqh = wq_all[s]
            woh = wo_all[s]
            for b in range(BQ):
                q = jnp.dot(xb16[b], wqh,
                            preferred_element_type=jnp.float32)
                ctx = []
                for hh in range(HG):
                    qh = q[:, hh * DH:(hh + 1) * DH].astype(jnp.bfloat16)
                    kth = kts[s, b, hh].astype(jnp.bfloat16)
                    sc = jnp.dot(qh, kth,
                                 preferred_element_type=jnp.float32) * 0.125
                    sc = jnp.where(mask, sc, -1e9)
                    e = jnp.exp(sc - jnp.max(sc, axis=1, keepdims=True))
                    w = (e / jnp.sum(e, axis=1, keepdims=True)
                         ).astype(jnp.bfloat16)
                    vth = vts[s, b, hh].astype(jnp.bfloat16)
                    ctx.append(lax.dot_general(
                        w, vth, (((1,), (1,)), ((), ())),
                        preferred_element_type=jnp.float32))
                ctxc = jnp.concatenate(ctx, axis=1).astype(jnp.bfloat16)
                contrib = jnp.dot(ctxc, woh,
                                  preferred_element_type=jnp.float32)
                if s == 0:
                    out_ref[b] = contrib
                else:
                    out_ref[b] = out_ref[b] + contrib

        for r in (a_wq, a_wo, b_wq, b_wo):
            r.start()
        compute_slot(0)
        for r in (a_wq, a_wo, b_wq, b_wo):
            r.wait()
        for r in (c_wq, c_wo, d_wq, d_wo):
            r.start()
        compute_slot(1)
        compute_slot(2)
        for r in (c_wq, c_wo, d_wq, d_wo):
            r.wait()
        compute_slot(3)

    out_shape = jax.ShapeDtypeStruct((BQ, SQ, DM), jnp.float32)
    return pl.pallas_call(
        body,
        out_shape=out_shape,
        in_specs=[
            pl.BlockSpec(memory_space=pltpu.VMEM),
            pl.BlockSpec(memory_space=pltpu.VMEM),
            pl.BlockSpec(memory_space=pl.ANY),
            pl.BlockSpec(memory_space=pl.ANY),
            pl.BlockSpec(memory_space=pltpu.VMEM),
        ],
        out_specs=pl.BlockSpec(memory_space=pltpu.VMEM),
        scratch_shapes=[
            pltpu.VMEM((N_DEV, DM, DQ), jnp.bfloat16),
            pltpu.VMEM((N_DEV, DQ, DM), jnp.bfloat16),
            pltpu.VMEM((N_DEV, BQ, HG, DH, SQ), jnp.float32),
            pltpu.VMEM((N_DEV, BQ, HG, DH, SQ), jnp.float32),
            pltpu.SemaphoreType.DMA((8,)),
            pltpu.SemaphoreType.DMA((8,)),
            pltpu.SemaphoreType.DMA((N_DEV, BQ)),
            pltpu.SemaphoreType.DMA((N_DEV, BQ)),
        ],
        compiler_params=pltpu.CompilerParams(collective_id=0),
    )(x, Wq, kT, vT, Wo)


# device time: 20046 ns/iter; 2.8158x vs baseline; 1.1210x over previous
import jax
import jax.numpy as jnp
from jax import lax
from jax.experimental import pallas as pl
from jax.experimental.pallas import tpu as pltpu

N_DEV = 4
BQ = 2
HG = 4
SQ = 256
DH = 64
DM = 512
DQ = 256

_MESH = pl.DeviceIdType.MESH


def kernel(x, Wq, K_ext, V_ext, Wo):
    b0 = lax.axis_index("i") * BQ
    kT = jnp.transpose(
        lax.dynamic_slice_in_dim(K_ext, b0, BQ, axis=0), (0, 2, 3, 1))
    vT = jnp.transpose(
        lax.dynamic_slice_in_dim(V_ext, b0, BQ, axis=0), (0, 2, 3, 1))

    def body(x_ref, wq_ref, kt_ref, vt_ref, wo_ref, out_ref,
             wq_all, wo_all, kts, vts,
             ssem, rsem, ksem, vsem):
        my = lax.axis_index("i")
        left = lax.rem(my + N_DEV - 1, N_DEV)
        right = lax.rem(my + 1, N_DEV)

        bar = pltpu.get_barrier_semaphore()
        for nbr in (left, right):
            pl.semaphore_signal(bar, inc=1, device_id=(nbr,),
                                device_id_type=_MESH)
        pl.semaphore_wait(bar, 2)

        slot_g = [my, left, right, lax.rem(my + 2, N_DEV)]

        kv_waits = []
        for s in range(N_DEV):
            g4 = slot_g[s] * HG
            group = []
            for b in range(BQ):
                ck = pltpu.make_async_copy(
                    kt_ref.at[b, pl.ds(g4, HG)],
                    kts.at[s, b], ksem.at[s, b])
                cv = pltpu.make_async_copy(
                    vt_ref.at[b, pl.ds(g4, HG)],
                    vts.at[s, b], vsem.at[s, b])
                ck.start()
                cv.start()
                group.append((ck, cv))
            kv_waits.append(group)

        wq_all[0] = wq_ref[...].astype(jnp.bfloat16)
        wo_all[0] = wo_ref[...].astype(jnp.bfloat16)

        def rcopy(i, src, dst, dev):
            return pltpu.make_async_remote_copy(
                src_ref=src, dst_ref=dst, send_sem=ssem.at[i],
                recv_sem=rsem.at[i], device_id=(dev,), device_id_type=_MESH)

        a_wq = rcopy(0, wq_all.at[0], wq_all.at[2], left)
        a_wo = rcopy(1, wo_all.at[0], wo_all.at[2], left)
        b_wq = rcopy(2, wq_all.at[0], wq_all.at[1], right)
        b_wo = rcopy(3, wo_all.at[0], wo_all.at[1], right)
        c_wq = rcopy(4, wq_all.at[2, pl.ds(0, DM // 2)],
                     wq_all.at[3, pl.ds(0, DM // 2)], left)
        c_wo = rcopy(5, wo_all.at[2, pl.ds(0, DQ // 2)],
                     wo_all.at[3, pl.ds(0, DQ // 2)], left)
        d_wq = rcopy(6, wq_all.at[1, pl.ds(DM // 2, DM // 2)],
                     wq_all.at[3, pl.ds(DM // 2, DM // 2)], right)
        d_wo = rcopy(7, wo_all.at[1, pl.ds(DQ // 2, DQ // 2)],
                     wo_all.at[3, pl.ds(DQ // 2, DQ // 2)], right)

        qb = lax.broadcasted_iota(jnp.int32, (SQ, SQ), 0) // 64
        kb = lax.broadcasted_iota(jnp.int32, (SQ, SQ), 1) // 64
        mask = (qb == kb) | ((kb % 4) == (qb % 4))

        xb16 = [x_ref[b].astype(jnp.bfloat16) for b in range(BQ)]
        acc = [None, None]

        def attn_slot(s):
            for ck, cv in kv_waits[s]:
                ck.wait()
                cv.wait()
            wqh = wq_all[s]
            ctxs = []
            for b in range(BQ):
                q = jnp.dot(xb16[b], wqh,
                            preferred_element_type=jnp.float32)
                ctx = []
                for hh in range(HG):
                    qh = q[:, hh * DH:(hh + 1) * DH].astype(jnp.bfloat16)
                    kth = kts[s, b, hh].astype(jnp.bfloat16)
                    sc = jnp.dot(qh, kth,
                                 preferred_element_type=jnp.float32) * 0.125
                    sc = jnp.where(mask, sc, -1e9)
                    e = jnp.exp(sc - jnp.max(sc, axis=1, keepdims=True))
                    w = (e / jnp.sum(e, axis=1, keepdims=True)
                         ).astype(jnp.bfloat16)
                    vth = vts[s, b, hh].astype(jnp.bfloat16)
                    ctx.append(lax.dot_general(
                        w, vth, (((1,), (1,)), ((), ())),
                        preferred_element_type=jnp.float32))
                ctxs.append(jnp.concatenate(ctx, axis=1).astype(jnp.bfloat16))
            return ctxs

        def out_slot(s, ctxs):
            woh = wo_all[s]
            for b in range(BQ):
                contrib = jnp.dot(ctxs[b], woh,
                                  preferred_element_type=jnp.float32)
                acc[b] = contrib if s == 0 else acc[b] + contrib

        for r in (a_wq, a_wo, b_wq, b_wo):
            r.start()
        out_slot(0, attn_slot(0))
        a_wq.wait()
        b_wq.wait()
        c_wq.start()
        d_wq.start()
        ctx1 = attn_slot(1)
        a_wo.wait()
        b_wo.wait()
        c_wo.start()
        d_wo.start()
        out_slot(1, ctx1)
        out_slot(2, attn_slot(2))
        c_wq.wait()
        d_wq.wait()
        ctx3 = attn_slot(3)
        c_wo.wait()
        d_wo.wait()
        out_slot(3, ctx3)
        for b in range(BQ):
            out_ref[b] = acc[b]

    out_shape = jax.ShapeDtypeStruct((BQ, SQ, DM), jnp.float32)
    return pl.pallas_call(
        body,
        out_shape=out_shape,
        in_specs=[
            pl.BlockSpec(memory_space=pltpu.VMEM),
            pl.BlockSpec(memory_space=pltpu.VMEM),
            pl.BlockSpec(memory_space=pl.ANY),
            pl.BlockSpec(memory_space=pl.ANY),
            pl.BlockSpec(memory_space=pltpu.VMEM),
        ],
        out_specs=pl.BlockSpec(memory_space=pltpu.VMEM),
        scratch_shapes=[
            pltpu.VMEM((N_DEV, DM, DQ), jnp.bfloat16),
            pltpu.VMEM((N_DEV, DQ, DM), jnp.bfloat16),
            pltpu.VMEM((N_DEV, BQ, HG, DH, SQ), jnp.float32),
            pltpu.VMEM((N_DEV, BQ, HG, DH, SQ), jnp.float32),
            pltpu.SemaphoreType.DMA((8,)),
            pltpu.SemaphoreType.DMA((8,)),
            pltpu.SemaphoreType.DMA((N_DEV, BQ)),
            pltpu.SemaphoreType.DMA((N_DEV, BQ)),
        ],
        compiler_params=pltpu.CompilerParams(collective_id=0),
    )(x, Wq, kT, vT, Wo)
